# baseline (speedup 1.0000x reference)
import functools

import jax
import jax.numpy as jnp
from jax import lax
from jax.experimental import pallas as pl
from jax.experimental.pallas import tpu as pltpu
from jax.experimental.pallas import tpu_sc as plsc

_INFO = plsc.get_sparse_core_info()
_NC = _INFO.num_cores
_NS = _INFO.num_subcores
_NW = _NC * _NS
_L = 16
_CW = 1024
_NCH = 32
_BCAP = 48
_MCAP = 1024


def _make_gather(dim, batch, n_rows):
    range_w = _NCH * _CW
    assert range_w * _NW >= n_rows
    n_phys = ((n_rows + 127) // 128) * 128
    max_start = n_phys - _CW
    mesh = plsc.VectorSubcoreMesh(core_axis_name="c", subcore_axis_name="s")

    @functools.partial(
        pl.kernel,
        mesh=mesh,
        out_type=jax.ShapeDtypeStruct((batch + 8, 128), jnp.float32),
        scratch_types=[
            pltpu.VMEM((batch // 2,), jnp.int32),
            pltpu.VMEM((_MCAP + _L,), jnp.int32),
            pltpu.VMEM((_MCAP + _L,), jnp.int32),
            pltpu.VMEM((_NCH, _BCAP), jnp.int32),
            pltpu.VMEM((_NCH, _BCAP), jnp.int32),
            pltpu.VMEM((dim, _CW), jnp.float32),
            pltpu.VMEM((dim, _CW), jnp.float32),
            pltpu.VMEM((_BCAP, 128), jnp.float32),
            pltpu.VMEM((_BCAP, 128), jnp.float32),
            pltpu.SemaphoreType.DMA,
            pltpu.SemaphoreType.DMA,
        ],
        compiler_params=pltpu.CompilerParams(needs_layout_passes=False),
    )
    def gather(idx_hbm, tab_hbm, img_hbm, idx_v, mi_v, mk_v, bi_v, bk_v,
               ring_a, ring_b, pad_a, pad_b, sem, wsem):
        wid = lax.axis_index("s") * _NC + lax.axis_index("c")
        range0 = wid * range_w
        lanes = lax.iota(jnp.int32, _L)

        def filt(half, off0):
            pltpu.sync_copy(
                idx_hbm.at[pl.ds(half * (batch // 2), batch // 2)], idx_v
            )

            def fbody(b, off):
                v16 = idx_v[pl.ds(b * _L, _L)]
                mask = lax.shift_right_logical(v16, 15) == wid
                plsc.store_compressed(
                    mi_v.at[pl.ds(off, _L)], v16, mask=mask
                )
                k16 = half * (batch // 2) + b * _L + lanes
                plsc.store_compressed(
                    mk_v.at[pl.ds(off, _L)], k16, mask=mask
                )
                return off + plsc.all_reduce_population_count(mask)[0]

            return lax.fori_loop(0, batch // (2 * _L), fbody, off0)

        n_mine = filt(1, filt(0, 0))

        for m in range(_NCH):
            for g in range(_BCAP // _L):
                bk_v[m, pl.ds(g * _L, _L)] = jnp.full(
                    (_L,), batch, jnp.int32
                )

        n_groups = lax.div(n_mine + (_L - 1), _L)

        def bbody(g, offs):
            i16 = mi_v[pl.ds(g * _L, _L)]
            k16 = mk_v[pl.ds(g * _L, _L)]
            valid = (g * _L + lanes) < n_mine
            bucket = lax.shift_right_logical(i16 - range0, 10)
            new_offs = []
            for m in range(_NCH):
                mask = valid & (bucket == m)
                off_m = offs[m]
                plsc.store_compressed(
                    bi_v.at[m, pl.ds(off_m, _L)], i16, mask=mask
                )
                plsc.store_compressed(
                    bk_v.at[m, pl.ds(off_m, _L)], k16, mask=mask
                )
                new_offs.append(
                    off_m + plsc.all_reduce_population_count(mask)[0]
                )
            return new_offs

        lax.fori_loop(0, n_groups, bbody, [0] * _NCH)

        def start_of(c):
            s = jnp.minimum(range0 + c * _CW, max_start)
            return pl.multiple_of(s, 128)

        def fire(c, ring):
            pltpu.async_copy(
                tab_hbm.at[:, pl.ds(start_of(c), _CW)], ring, sem
            )

        def drain_fetch(ring):
            pltpu.make_async_copy(
                tab_hbm.at[:, pl.ds(0, _CW)], ring, sem
            ).wait()

        def extract(c, ring, pad):
            start = start_of(c)
            for g in range(_BCAP // _L):
                i16 = bi_v[c, pl.ds(g * _L, _L)]
                l16 = lax.bitwise_and(i16 - start, _CW - 1)
                for j in range(dim):
                    vals = plsc.load_gather(
                        ring, [jnp.full((_L,), j, jnp.int32), l16]
                    )
                    plsc.store_scatter(
                        pad,
                        [g * _L + lanes, jnp.full((_L,), j, jnp.int32)],
                        vals,
                    )
            pltpu.async_copy(pad, img_hbm.at[bk_v.at[c]], wsem)

        def drain_scatter(pad):
            pltpu.make_async_copy(
                img_hbm.at[pl.ds(0, _BCAP)], pad, wsem
            ).wait()

        fire(0, ring_a)

        def body(p, carry):
            c = p * 2
            fire(c + 1, ring_b)
            drain_fetch(ring_a)

            @pl.when(p > 0)
            def _a():
                drain_scatter(pad_a)

            extract(c, ring_a, pad_a)

            @pl.when(p + 1 < _NCH // 2)
            def _f():
                fire(c + 2, ring_a)

            drain_fetch(ring_b)

            @pl.when(p > 0)
            def _b():
                drain_scatter(pad_b)

            extract(c + 1, ring_b, pad_b)
            return carry

        lax.fori_loop(0, _NCH // 2, body, 0)
        drain_scatter(pad_a)
        drain_scatter(pad_b)

    return gather


def _make_compact(dim, batch):
    b_per_w = batch // _NW
    mesh = plsc.VectorSubcoreMesh(core_axis_name="c", subcore_axis_name="s")

    @functools.partial(
        pl.kernel,
        mesh=mesh,
        out_type=jax.ShapeDtypeStruct((dim, batch), jnp.float32),
        scratch_types=[
            pltpu.VMEM((batch // _NW // 2, 128), jnp.float32),
            pltpu.VMEM((dim, batch // _NW), jnp.float32),
            pltpu.SemaphoreType.DMA,
        ],
        compiler_params=pltpu.CompilerParams(needs_layout_passes=False),
    )
    def compact(img_hbm, out_hbm, chunk_v, outt_v, sem):
        wid = lax.axis_index("s") * _NC + lax.axis_index("c")
        base = wid * b_per_w
        lanes = lax.iota(jnp.int32, _L)
        half = b_per_w // 2
        for h in range(2):
            pltpu.sync_copy(
                img_hbm.at[pl.ds(base + h * half, half)], chunk_v
            )
            for j in range(dim):
                jsplat = jnp.full((_L,), j, jnp.int32)
                for g in range(half // _L):
                    vals = plsc.load_gather(
                        chunk_v, [g * _L + lanes, jsplat]
                    )
                    outt_v[j, pl.ds(h * half + g * _L, _L)] = vals
        pltpu.sync_copy(outt_v, out_hbm.at[:, pl.ds(base, b_per_w)])

    return compact


@jax.jit
def kernel(skill_id, emb_weight):
    batch = skill_id.shape[0]
    n_rows, dim = emb_weight.shape
    img = _make_gather(dim, batch, n_rows)(
        skill_id.astype(jnp.int32), emb_weight.T
    )
    out_t = _make_compact(dim, batch)(img)
    return out_t.T


# speedup vs baseline: 13.2362x; 13.2362x over previous
import functools

import jax
import jax.numpy as jnp
from jax import lax
from jax.experimental import pallas as pl
from jax.experimental.pallas import tpu as pltpu
from jax.experimental.pallas import tpu_sc as plsc

_INFO = plsc.get_sparse_core_info()
_NC = _INFO.num_cores
_NS = _INFO.num_subcores
_NW = _NC * _NS
_L = 16


def _make_lookup(dim, batch, n_rows):
    assert batch % (_NW * _L) == 0
    b_per_w = batch // _NW
    n_blocks = b_per_w // _L
    mesh = plsc.VectorSubcoreMesh(core_axis_name="c", subcore_axis_name="s")

    @functools.partial(
        pl.kernel,
        mesh=mesh,
        out_type=jax.ShapeDtypeStruct((dim, batch), jnp.float32),
        scratch_types=[
            pltpu.VMEM((b_per_w + _L,), jnp.int32),
            pltpu.VMEM((_L, dim, 128), jnp.float32),
            pltpu.VMEM((dim, b_per_w), jnp.float32),
            pltpu.SemaphoreType.DMA,
        ],
        compiler_params=pltpu.CompilerParams(needs_layout_passes=False),
    )
    def lookup(idx_hbm, tab_hbm, out_hbm, idx_v, ring_v, outt_v, sem):
        wid = lax.axis_index("s") * _NC + lax.axis_index("c")
        base = wid * b_per_w
        pltpu.sync_copy(
            idx_hbm.at[pl.ds(base, b_per_w)], idx_v.at[pl.ds(0, b_per_w)]
        )

        lanes = lax.iota(jnp.int32, _L)

        def fire(v, j):
            v = jnp.clip(v, 0, n_rows - 1)
            col0 = pl.multiple_of(
                lax.shift_left(lax.shift_right_logical(v, 7), 7), 128
            )
            pltpu.async_copy(
                tab_hbm.at[:, pl.ds(col0, 128)], ring_v.at[j], sem
            )

        def drain1(j):
            pltpu.make_async_copy(
                tab_hbm.at[:, pl.ds(0, 128)], ring_v.at[j], sem
            ).wait()

        v0 = idx_v[pl.ds(0, _L)]
        for j in range(_L):
            fire(v0[j], j)

        def body(b, carry):
            k0 = b * _L
            vnext = idx_v[pl.ds(k0 + _L, _L)]
            for j in range(_L):
                drain1(j)
                lsplat = plsc.load_gather(
                    idx_v, [jnp.full((_L,), k0 + j, jnp.int32)]
                )
                lsplat = lax.bitwise_and(lsplat, 127)
                ksplat = jnp.full((_L,), k0 + j, jnp.int32)
                jsplat = jnp.full((_L,), j, jnp.int32)
                for h in range(dim // _L):
                    vals = plsc.load_gather(
                        ring_v, [jsplat, lanes + h * _L, lsplat]
                    )
                    plsc.store_scatter(
                        outt_v, [lanes + h * _L, ksplat], vals
                    )
                fire(vnext[j], j)
            return carry

        lax.fori_loop(0, n_blocks, body, 0)
        for j in range(_L):
            drain1(j)

        pltpu.sync_copy(outt_v, out_hbm.at[:, pl.ds(base, b_per_w)])

    return lookup


@jax.jit
def kernel(skill_id, emb_weight):
    batch = skill_id.shape[0]
    n_rows, dim = emb_weight.shape
    out_t = _make_lookup(dim, batch, n_rows)(
        skill_id.astype(jnp.int32), emb_weight.T
    )
    return out_t.T
